# Initial kernel scaffold; baseline (speedup 1.0000x reference)
import jax
import jax.numpy as jnp
from jax.experimental import pallas as pl


def kernel(x, edge_index, W1, b1, W2, b2, Wmu, bmu, Wlv, blv):
    raise NotImplementedError("write your pallas kernel here")


# speedup vs baseline: 21.9931x; 21.9931x over previous
import functools

import jax
import jax.numpy as jnp
from jax import lax
from jax.experimental import pallas as pl
from jax.experimental.pallas import tpu as pltpu
from jax.experimental.pallas import tpu_sc as plsc

N = 10000
E = 320000
D = 128
DL = 64

NC = 2
NS = 16
NW = NC * NS

CHUNK = 128
CH = -(-E // (NW * CHUNK))
T = CH * CHUNK
E_PAD = NW * T
ROWS_PER_TILE = 632
N_PAD = NS * ROWS_PER_TILE

_mesh = plsc.VectorSubcoreMesh(core_axis_name="c", subcore_axis_name="s")


N_PAD_DEG = NS * 640


@functools.partial(
    pl.kernel,
    out_type=jax.ShapeDtypeStruct((NC, N_PAD_DEG), jnp.float32),
    mesh=_mesh,
    scratch_types=[
        pltpu.VMEM((CH, CHUNK), jnp.int32),
        pltpu.VMEM((CHUNK,), jnp.float32),
        pltpu.VMEM((CHUNK,), jnp.float32),
        pltpu.VMEM_SHARED((N_PAD_DEG,), jnp.float32),
    ],
)
def _deg_kernel(dst_hbm, out_hbm, dst_v, ones_v, zeros_v, acc_s):
    cid = lax.axis_index("c")
    sid = lax.axis_index("s")
    wid = sid * NC + cid
    pltpu.sync_copy(dst_hbm.at[wid], dst_v)

    def fill_body(i, _):
        ones_v[pl.ds(i * 16, 16)] = jnp.ones((16,), jnp.float32)
        zeros_v[pl.ds(i * 16, 16)] = jnp.zeros((16,), jnp.float32)
        return 0

    lax.fori_loop(0, CHUNK // 16, fill_body, 0)

    base = sid * 640
    for k in range(5):
        pltpu.sync_copy(zeros_v, acc_s.at[pl.ds(base + k * CHUNK, CHUNK)])
    plsc.subcore_barrier()

    def hist_body(j, _):
        pltpu.sync_copy(ones_v, acc_s.at[dst_v.at[j]], add=True)
        return 0

    lax.fori_loop(0, CH, hist_body, 0)
    plsc.subcore_barrier()

    pltpu.sync_copy(acc_s.at[pl.ds(base, 640)],
                    out_hbm.at[cid].at[pl.ds(base, 640)])


@functools.partial(
    pl.kernel,
    out_type=jax.ShapeDtypeStruct((NC, N_PAD, D), jnp.float32),
    mesh=_mesh,
    scratch_types=[
        pltpu.VMEM((CH, CHUNK), jnp.int32),
        pltpu.VMEM((CH, CHUNK), jnp.int32),
        pltpu.VMEM((CHUNK, D), jnp.float32),
        pltpu.VMEM_SHARED((N_PAD, D), jnp.float32),
        pltpu.SemaphoreType.DMA,
    ],
)
def _agg_kernel(g_hbm, src_hbm, dst_hbm, out_hbm,
                src_v, dst_v, rows0_v, acc_s, sem0):
    cid = lax.axis_index("c")
    sid = lax.axis_index("s")
    wid = sid * NC + cid
    pltpu.sync_copy(src_hbm.at[wid], src_v)
    pltpu.sync_copy(dst_hbm.at[wid], dst_v)

    zeros = jnp.zeros((16,), jnp.float32)

    def zero_body(i, _):
        rows0_v[i // (D // 16), pl.ds((i % (D // 16)) * 16, 16)] = zeros
        return 0

    lax.fori_loop(0, CHUNK * (D // 16), zero_body, 0)

    base = sid * ROWS_PER_TILE
    nfull = ROWS_PER_TILE // CHUNK
    rem = ROWS_PER_TILE - nfull * CHUNK
    for k in range(nfull):
        pltpu.sync_copy(rows0_v, acc_s.at[pl.ds(base + k * CHUNK, CHUNK)])
    if rem:
        pltpu.sync_copy(rows0_v.at[pl.ds(0, rem)],
                        acc_s.at[pl.ds(base + nfull * CHUNK, rem)])
    plsc.subcore_barrier()

    def chunk_body(j, _):
        pltpu.async_copy(g_hbm.at[src_v.at[j]], rows0_v, sem0).wait()
        pltpu.sync_copy(rows0_v, acc_s.at[dst_v.at[j]], add=True)
        return 0

    lax.fori_loop(0, CH, chunk_body, 0)
    plsc.subcore_barrier()

    pltpu.sync_copy(acc_s.at[pl.ds(base, ROWS_PER_TILE)],
                    out_hbm.at[cid].at[pl.ds(base, ROWS_PER_TILE)])


_R = 2048
_G = -(-N // _R)


def _dinv_body(degp_ref, x_ref, dinv_ref, g_ref):
    deg = 1.0 + degp_ref[0, :] + degp_ref[1, :]
    di = lax.rsqrt(deg)[:, None]
    dinv_ref[...] = di
    g_ref[...] = x_ref[...] * di


_dinv_call = pl.pallas_call(
    _dinv_body,
    grid=(_G,),
    in_specs=[
        pl.BlockSpec((NC, _R), lambda i: (0, i)),
        pl.BlockSpec((_R, D), lambda i: (i, 0)),
    ],
    out_specs=[
        pl.BlockSpec((_R, 1), lambda i: (i, 0)),
        pl.BlockSpec((_R, D), lambda i: (i, 0)),
    ],
    out_shape=[
        jax.ShapeDtypeStruct((N, 1), jnp.float32),
        jax.ShapeDtypeStruct((N, D), jnp.float32),
    ],
)


def _layer_body(p_ref, v_ref, dinv_ref, w_ref, b_ref, h_ref, g_ref):
    di = dinv_ref[...]
    t = p_ref[0] + p_ref[1]
    u = di * t + (di * di) * v_ref[...]
    h = jnp.dot(u, w_ref[...], preferred_element_type=jnp.float32)
    h = jnp.maximum(h + b_ref[...], 0.0)
    h_ref[...] = h
    g_ref[...] = h * di


_layer_call = pl.pallas_call(
    _layer_body,
    grid=(_G,),
    in_specs=[
        pl.BlockSpec((NC, _R, D), lambda i: (0, i, 0)),
        pl.BlockSpec((_R, D), lambda i: (i, 0)),
        pl.BlockSpec((_R, 1), lambda i: (i, 0)),
        pl.BlockSpec((D, D), lambda i: (0, 0)),
        pl.BlockSpec((1, D), lambda i: (0, 0)),
    ],
    out_specs=[
        pl.BlockSpec((_R, D), lambda i: (i, 0)),
        pl.BlockSpec((_R, D), lambda i: (i, 0)),
    ],
    out_shape=[
        jax.ShapeDtypeStruct((N, D), jnp.float32),
        jax.ShapeDtypeStruct((N, D), jnp.float32),
    ],
)


def _head_body(p_ref, v_ref, dinv_ref, wmu_ref, bmu_ref, wlv_ref, blv_ref,
               mu_ref, lv_ref):
    di = dinv_ref[...]
    t = p_ref[0] + p_ref[1]
    a = di * t + (di * di) * v_ref[...]
    mu_ref[...] = jnp.dot(a, wmu_ref[...],
                          preferred_element_type=jnp.float32) + bmu_ref[...]
    lv_ref[...] = jnp.dot(a, wlv_ref[...],
                          preferred_element_type=jnp.float32) + blv_ref[...]


_head_call = pl.pallas_call(
    _head_body,
    grid=(_G,),
    in_specs=[
        pl.BlockSpec((NC, _R, D), lambda i: (0, i, 0)),
        pl.BlockSpec((_R, D), lambda i: (i, 0)),
        pl.BlockSpec((_R, 1), lambda i: (i, 0)),
        pl.BlockSpec((D, DL), lambda i: (0, 0)),
        pl.BlockSpec((1, DL), lambda i: (0, 0)),
        pl.BlockSpec((D, DL), lambda i: (0, 0)),
        pl.BlockSpec((1, DL), lambda i: (0, 0)),
    ],
    out_specs=[
        pl.BlockSpec((_R, DL), lambda i: (i, 0)),
        pl.BlockSpec((_R, DL), lambda i: (i, 0)),
    ],
    out_shape=[
        jax.ShapeDtypeStruct((N, DL), jnp.float32),
        jax.ShapeDtypeStruct((N, DL), jnp.float32),
    ],
)


@jax.jit
def kernel(x, edge_index, W1, b1, W2, b2, Wmu, bmu, Wlv, blv):
    src = edge_index[0]
    dst = edge_index[1]

    pad = E_PAD - E
    pad_ids = lax.iota(jnp.int32, pad)
    src_p = jnp.concatenate([src, pad_ids % N])
    dst_p = jnp.concatenate([dst, N + (pad_ids % 16)])
    src_rows = src_p.reshape(NW, CH, CHUNK)
    dst_rows = dst_p.reshape(NW, CH, CHUNK)

    degp = _deg_kernel(dst_rows)
    dinv, g = _dinv_call(degp, x)

    p = _agg_kernel(g, src_rows, dst_rows)
    h1, g1 = _layer_call(p, x, dinv, W1, b1.reshape(1, D))

    p = _agg_kernel(g1, src_rows, dst_rows)
    h2, g2 = _layer_call(p, h1, dinv, W2, b2.reshape(1, D))

    p = _agg_kernel(g2, src_rows, dst_rows)
    mu, logvar = _head_call(p, h2, dinv, Wmu, bmu.reshape(1, DL),
                            Wlv, blv.reshape(1, DL))
    return (mu, logvar)


# speedup vs baseline: 32.7157x; 1.4875x over previous
import functools

import jax
import jax.numpy as jnp
from jax import lax
from jax.experimental import pallas as pl
from jax.experimental.pallas import tpu as pltpu
from jax.experimental.pallas import tpu_sc as plsc

N = 10000
E = 320000
D = 128
DL = 64

NC = 2
NS = 16
NW = NC * NS

CHUNK = 128
CH = -(-E // (NW * CHUNK))
T = CH * CHUNK
E_PAD = NW * T
ROWS_PER_TILE = 632
N_PAD = NS * ROWS_PER_TILE

_mesh = plsc.VectorSubcoreMesh(core_axis_name="c", subcore_axis_name="s")


N_PAD_DEG = NS * 640


@functools.partial(
    pl.kernel,
    out_type=jax.ShapeDtypeStruct((NC, N_PAD_DEG), jnp.float32),
    mesh=_mesh,
    scratch_types=[
        pltpu.VMEM((CH, CHUNK), jnp.int32),
        pltpu.VMEM((CHUNK,), jnp.float32),
        pltpu.VMEM((CHUNK,), jnp.float32),
        pltpu.VMEM_SHARED((N_PAD_DEG,), jnp.float32),
    ],
)
def _deg_kernel(dst_hbm, out_hbm, dst_v, ones_v, zeros_v, acc_s):
    cid = lax.axis_index("c")
    sid = lax.axis_index("s")
    wid = sid * NC + cid
    pltpu.sync_copy(dst_hbm.at[wid], dst_v)

    def fill_body(i, _):
        ones_v[pl.ds(i * 16, 16)] = jnp.ones((16,), jnp.float32)
        zeros_v[pl.ds(i * 16, 16)] = jnp.zeros((16,), jnp.float32)
        return 0

    lax.fori_loop(0, CHUNK // 16, fill_body, 0)

    base = sid * 640
    for k in range(5):
        pltpu.sync_copy(zeros_v, acc_s.at[pl.ds(base + k * CHUNK, CHUNK)])
    plsc.subcore_barrier()

    def hist_body(j, _):
        pltpu.sync_copy(ones_v, acc_s.at[dst_v.at[j]], add=True)
        return 0

    lax.fori_loop(0, CH, hist_body, 0)
    plsc.subcore_barrier()

    pltpu.sync_copy(acc_s.at[pl.ds(base, 640)],
                    out_hbm.at[cid].at[pl.ds(base, 640)])


@functools.partial(
    pl.kernel,
    out_type=jax.ShapeDtypeStruct((NC, N_PAD, D), jnp.float32),
    mesh=_mesh,
    scratch_types=[
        pltpu.VMEM((CH, CHUNK), jnp.int32),
        pltpu.VMEM((2, CHUNK), jnp.int32),
        pltpu.VMEM((CHUNK, D), jnp.float32),
        pltpu.VMEM((CHUNK, D), jnp.float32),
        pltpu.VMEM_SHARED((N_PAD, D), jnp.float32),
        pltpu.SemaphoreType.DMA,
        pltpu.SemaphoreType.DMA,
        pltpu.SemaphoreType.DMA,
        pltpu.SemaphoreType.DMA,
    ],
)
def _agg_kernel(g_hbm, src_hbm, dst_hbm, out_hbm,
                src_v, dstb_v, rows0_v, rows1_v, acc_s,
                semg0, semg1, semd0, semd1):
    cid = lax.axis_index("c")
    sid = lax.axis_index("s")
    wid = sid * NC + cid
    pltpu.sync_copy(src_hbm.at[wid], src_v)

    zeros = jnp.zeros((16,), jnp.float32)

    def zero_body(i, _):
        rows0_v[i // (D // 16), pl.ds((i % (D // 16)) * 16, 16)] = zeros
        return 0

    lax.fori_loop(0, CHUNK * (D // 16), zero_body, 0)

    base = sid * ROWS_PER_TILE
    nfull = ROWS_PER_TILE // CHUNK
    rem = ROWS_PER_TILE - nfull * CHUNK
    for k in range(nfull):
        pltpu.sync_copy(rows0_v, acc_s.at[pl.ds(base + k * CHUNK, CHUNK)])
    if rem:
        pltpu.sync_copy(rows0_v.at[pl.ds(0, rem)],
                        acc_s.at[pl.ds(base + nfull * CHUNK, rem)])
    plsc.subcore_barrier()

    bufs = ((rows0_v, semg0, semd0), (rows1_v, semg1, semd1))

    def issue(j, k):
        rows, semg, semd = bufs[k]
        pltpu.async_copy(g_hbm.at[src_v.at[j]], rows, semg)
        pltpu.async_copy(dst_hbm.at[wid].at[j], dstb_v.at[k], semd)

    issue(0, 0)

    def chunk_body(j, _):
        for k in (0, 1):
            @pl.when(j % 2 == k)
            def _():
                rows, semg, semd = bufs[k]

                @pl.when(j + 1 < CH)
                def _():
                    issue(j + 1, 1 - k)

                pltpu.make_async_copy(
                    g_hbm.at[src_v.at[j]], rows, semg).wait()
                pltpu.make_async_copy(
                    dst_hbm.at[wid].at[j], dstb_v.at[k], semd).wait()
                pltpu.sync_copy(rows, acc_s.at[dstb_v.at[k]], add=True)

        return 0

    lax.fori_loop(0, CH, chunk_body, 0)
    plsc.subcore_barrier()

    pltpu.sync_copy(acc_s.at[pl.ds(base, ROWS_PER_TILE)],
                    out_hbm.at[cid].at[pl.ds(base, ROWS_PER_TILE)])


_R = 2048
_G = -(-N // _R)


def _dinv_body(degp_ref, x_ref, dinv_ref, g_ref):
    deg = 1.0 + degp_ref[0, :] + degp_ref[1, :]
    di = lax.rsqrt(deg)[:, None]
    dinv_ref[...] = di
    g_ref[...] = x_ref[...] * di


_dinv_call = pl.pallas_call(
    _dinv_body,
    grid=(_G,),
    in_specs=[
        pl.BlockSpec((NC, _R), lambda i: (0, i)),
        pl.BlockSpec((_R, D), lambda i: (i, 0)),
    ],
    out_specs=[
        pl.BlockSpec((_R, 1), lambda i: (i, 0)),
        pl.BlockSpec((_R, D), lambda i: (i, 0)),
    ],
    out_shape=[
        jax.ShapeDtypeStruct((N, 1), jnp.float32),
        jax.ShapeDtypeStruct((N, D), jnp.float32),
    ],
)


def _layer_body(p_ref, v_ref, dinv_ref, w_ref, b_ref, h_ref, g_ref):
    di = dinv_ref[...]
    t = p_ref[0] + p_ref[1]
    u = di * t + (di * di) * v_ref[...]
    h = jnp.dot(u, w_ref[...], preferred_element_type=jnp.float32)
    h = jnp.maximum(h + b_ref[...], 0.0)
    h_ref[...] = h
    g_ref[...] = h * di


_layer_call = pl.pallas_call(
    _layer_body,
    grid=(_G,),
    in_specs=[
        pl.BlockSpec((NC, _R, D), lambda i: (0, i, 0)),
        pl.BlockSpec((_R, D), lambda i: (i, 0)),
        pl.BlockSpec((_R, 1), lambda i: (i, 0)),
        pl.BlockSpec((D, D), lambda i: (0, 0)),
        pl.BlockSpec((1, D), lambda i: (0, 0)),
    ],
    out_specs=[
        pl.BlockSpec((_R, D), lambda i: (i, 0)),
        pl.BlockSpec((_R, D), lambda i: (i, 0)),
    ],
    out_shape=[
        jax.ShapeDtypeStruct((N, D), jnp.float32),
        jax.ShapeDtypeStruct((N, D), jnp.float32),
    ],
)


def _head_body(p_ref, v_ref, dinv_ref, wmu_ref, bmu_ref, wlv_ref, blv_ref,
               mu_ref, lv_ref):
    di = dinv_ref[...]
    t = p_ref[0] + p_ref[1]
    a = di * t + (di * di) * v_ref[...]
    mu_ref[...] = jnp.dot(a, wmu_ref[...],
                          preferred_element_type=jnp.float32) + bmu_ref[...]
    lv_ref[...] = jnp.dot(a, wlv_ref[...],
                          preferred_element_type=jnp.float32) + blv_ref[...]


_head_call = pl.pallas_call(
    _head_body,
    grid=(_G,),
    in_specs=[
        pl.BlockSpec((NC, _R, D), lambda i: (0, i, 0)),
        pl.BlockSpec((_R, D), lambda i: (i, 0)),
        pl.BlockSpec((_R, 1), lambda i: (i, 0)),
        pl.BlockSpec((D, DL), lambda i: (0, 0)),
        pl.BlockSpec((1, DL), lambda i: (0, 0)),
        pl.BlockSpec((D, DL), lambda i: (0, 0)),
        pl.BlockSpec((1, DL), lambda i: (0, 0)),
    ],
    out_specs=[
        pl.BlockSpec((_R, DL), lambda i: (i, 0)),
        pl.BlockSpec((_R, DL), lambda i: (i, 0)),
    ],
    out_shape=[
        jax.ShapeDtypeStruct((N, DL), jnp.float32),
        jax.ShapeDtypeStruct((N, DL), jnp.float32),
    ],
)


@jax.jit
def kernel(x, edge_index, W1, b1, W2, b2, Wmu, bmu, Wlv, blv):
    src = edge_index[0]
    dst = edge_index[1]

    pad = E_PAD - E
    pad_ids = lax.iota(jnp.int32, pad)
    src_p = jnp.concatenate([src, pad_ids % N])
    dst_p = jnp.concatenate([dst, N + (pad_ids % 16)])
    src_rows = src_p.reshape(NW, CH, CHUNK)
    dst_rows = dst_p.reshape(NW, CH, CHUNK)

    degp = _deg_kernel(dst_rows)
    dinv, g = _dinv_call(degp, x)

    p = _agg_kernel(g, src_rows, dst_rows)
    h1, g1 = _layer_call(p, x, dinv, W1, b1.reshape(1, D))

    p = _agg_kernel(g1, src_rows, dst_rows)
    h2, g2 = _layer_call(p, h1, dinv, W2, b2.reshape(1, D))

    p = _agg_kernel(g2, src_rows, dst_rows)
    mu, logvar = _head_call(p, h2, dinv, Wmu, bmu.reshape(1, DL),
                            Wlv, blv.reshape(1, DL))
    return (mu, logvar)


# speedup vs baseline: 32.8676x; 1.0046x over previous
import functools

import jax
import jax.numpy as jnp
from jax import lax
from jax.experimental import pallas as pl
from jax.experimental.pallas import tpu as pltpu
from jax.experimental.pallas import tpu_sc as plsc

N = 10000
E = 320000
D = 128
DL = 64

NC = 2
NS = 16
NW = NC * NS

CHUNK = 128
CH = -(-E // (NW * CHUNK))
T = CH * CHUNK
E_PAD = NW * T
ROWS_PER_TILE = 632
N_PAD = NS * ROWS_PER_TILE

_mesh = plsc.VectorSubcoreMesh(core_axis_name="c", subcore_axis_name="s")


N_PAD_DEG = NS * 640


@functools.partial(
    pl.kernel,
    out_type=jax.ShapeDtypeStruct((NC, N_PAD_DEG), jnp.float32),
    mesh=_mesh,
    scratch_types=[
        pltpu.VMEM((CH, CHUNK), jnp.int32),
        pltpu.VMEM((CHUNK,), jnp.float32),
        pltpu.VMEM((CHUNK,), jnp.float32),
        pltpu.VMEM_SHARED((N_PAD_DEG,), jnp.float32),
    ],
)
def _deg_kernel(dst_hbm, out_hbm, dst_v, ones_v, zeros_v, acc_s):
    cid = lax.axis_index("c")
    sid = lax.axis_index("s")
    wid = sid * NC + cid
    pltpu.sync_copy(dst_hbm.at[wid], dst_v)

    def fill_body(i, _):
        ones_v[pl.ds(i * 16, 16)] = jnp.ones((16,), jnp.float32)
        zeros_v[pl.ds(i * 16, 16)] = jnp.zeros((16,), jnp.float32)
        return 0

    lax.fori_loop(0, CHUNK // 16, fill_body, 0)

    base = sid * 640
    for k in range(5):
        pltpu.sync_copy(zeros_v, acc_s.at[pl.ds(base + k * CHUNK, CHUNK)])
    plsc.subcore_barrier()

    def hist_body(j, _):
        pltpu.sync_copy(ones_v, acc_s.at[dst_v.at[j]], add=True)
        return 0

    lax.fori_loop(0, CH, hist_body, 0)
    plsc.subcore_barrier()

    pltpu.sync_copy(acc_s.at[pl.ds(base, 640)],
                    out_hbm.at[cid].at[pl.ds(base, 640)])


@functools.partial(
    pl.kernel,
    out_type=jax.ShapeDtypeStruct((NC, N_PAD, D), jnp.float32),
    mesh=_mesh,
    scratch_types=[
        pltpu.VMEM((CH, CHUNK), jnp.int32),
        pltpu.VMEM((2, CHUNK), jnp.int32),
        pltpu.VMEM((CHUNK, D), jnp.float32),
        pltpu.VMEM((CHUNK, D), jnp.float32),
        pltpu.VMEM_SHARED((N_PAD, D), jnp.float32),
        pltpu.SemaphoreType.DMA,
        pltpu.SemaphoreType.DMA,
        pltpu.SemaphoreType.DMA,
        pltpu.SemaphoreType.DMA,
    ],
)
def _agg_kernel(g_hbm, src_hbm, dst_hbm, out_hbm,
                src_v, dstb_v, rows0_v, rows1_v, acc_s,
                semg0, semg1, semd0, semd1):
    cid = lax.axis_index("c")
    sid = lax.axis_index("s")
    wid = sid * NC + cid
    pltpu.sync_copy(src_hbm.at[wid], src_v)

    zeros = jnp.zeros((16,), jnp.float32)

    def zero_body(i, _):
        rows0_v[i // (D // 16), pl.ds((i % (D // 16)) * 16, 16)] = zeros
        return 0

    lax.fori_loop(0, CHUNK * (D // 16), zero_body, 0)

    base = sid * ROWS_PER_TILE
    nfull = ROWS_PER_TILE // CHUNK
    rem = ROWS_PER_TILE - nfull * CHUNK
    for k in range(nfull):
        pltpu.async_copy(rows0_v, acc_s.at[pl.ds(base + k * CHUNK, CHUNK)],
                         semg0)
    pltpu.async_copy(rows0_v.at[pl.ds(0, rem)],
                     acc_s.at[pl.ds(base + nfull * CHUNK, rem)], semg1)
    for k in range(nfull):
        pltpu.make_async_copy(
            rows0_v, acc_s.at[pl.ds(base + k * CHUNK, CHUNK)], semg0).wait()
    pltpu.make_async_copy(
        rows0_v.at[pl.ds(0, rem)],
        acc_s.at[pl.ds(base + nfull * CHUNK, rem)], semg1).wait()
    plsc.subcore_barrier()

    bufs = ((rows0_v, semg0, semd0), (rows1_v, semg1, semd1))

    def issue(j, k):
        rows, semg, semd = bufs[k]
        pltpu.async_copy(g_hbm.at[src_v.at[j]], rows, semg)
        pltpu.async_copy(dst_hbm.at[wid].at[j], dstb_v.at[k], semd)

    issue(0, 0)

    def chunk_body(j, _):
        for k in (0, 1):
            @pl.when(j % 2 == k)
            def _():
                rows, semg, semd = bufs[k]

                @pl.when(j + 1 < CH)
                def _():
                    issue(j + 1, 1 - k)

                pltpu.make_async_copy(
                    g_hbm.at[src_v.at[j]], rows, semg).wait()
                pltpu.make_async_copy(
                    dst_hbm.at[wid].at[j], dstb_v.at[k], semd).wait()
                pltpu.sync_copy(rows, acc_s.at[dstb_v.at[k]], add=True)

        return 0

    lax.fori_loop(0, CH, chunk_body, 0)
    plsc.subcore_barrier()

    pltpu.sync_copy(acc_s.at[pl.ds(base, ROWS_PER_TILE)],
                    out_hbm.at[cid].at[pl.ds(base, ROWS_PER_TILE)])


_R = 2048
_G = -(-N // _R)


def _dinv_body(degp_ref, x_ref, dinv_ref, g_ref):
    deg = 1.0 + degp_ref[0, :] + degp_ref[1, :]
    di = lax.rsqrt(deg)[:, None]
    dinv_ref[...] = di
    g_ref[...] = x_ref[...] * di


_dinv_call = pl.pallas_call(
    _dinv_body,
    grid=(_G,),
    in_specs=[
        pl.BlockSpec((NC, _R), lambda i: (0, i)),
        pl.BlockSpec((_R, D), lambda i: (i, 0)),
    ],
    out_specs=[
        pl.BlockSpec((_R, 1), lambda i: (i, 0)),
        pl.BlockSpec((_R, D), lambda i: (i, 0)),
    ],
    out_shape=[
        jax.ShapeDtypeStruct((N, 1), jnp.float32),
        jax.ShapeDtypeStruct((N, D), jnp.float32),
    ],
)


def _mm1_body(x_ref, w_ref, y_ref):
    y_ref[...] = jnp.dot(x_ref[...], w_ref[...],
                         preferred_element_type=jnp.float32)


_mm1_call = pl.pallas_call(
    _mm1_body,
    grid=(_G,),
    in_specs=[
        pl.BlockSpec((_R, D), lambda i: (i, 0)),
        pl.BlockSpec((D, D), lambda i: (0, 0)),
    ],
    out_specs=pl.BlockSpec((_R, D), lambda i: (i, 0)),
    out_shape=jax.ShapeDtypeStruct((N, D), jnp.float32),
)


def _layer1_body(p_ref, y_ref, dinv_ref, b_ref, h_ref, g_ref):
    di = dinv_ref[...]
    t = p_ref[0] + p_ref[1]
    u = di * t + (di * di) * y_ref[...] + b_ref[...]
    h = jnp.maximum(u, 0.0)
    h_ref[...] = h
    g_ref[...] = h * di


_layer1_call = pl.pallas_call(
    _layer1_body,
    grid=(_G,),
    in_specs=[
        pl.BlockSpec((NC, _R, D), lambda i: (0, i, 0)),
        pl.BlockSpec((_R, D), lambda i: (i, 0)),
        pl.BlockSpec((_R, 1), lambda i: (i, 0)),
        pl.BlockSpec((1, D), lambda i: (0, 0)),
    ],
    out_specs=[
        pl.BlockSpec((_R, D), lambda i: (i, 0)),
        pl.BlockSpec((_R, D), lambda i: (i, 0)),
    ],
    out_shape=[
        jax.ShapeDtypeStruct((N, D), jnp.float32),
        jax.ShapeDtypeStruct((N, D), jnp.float32),
    ],
)


def _layer_body(p_ref, v_ref, dinv_ref, w_ref, b_ref, h_ref, g_ref):
    di = dinv_ref[...]
    t = p_ref[0] + p_ref[1]
    u = di * t + (di * di) * v_ref[...]
    h = jnp.dot(u, w_ref[...], preferred_element_type=jnp.float32)
    h = jnp.maximum(h + b_ref[...], 0.0)
    h_ref[...] = h
    g_ref[...] = h * di


_layer_call = pl.pallas_call(
    _layer_body,
    grid=(_G,),
    in_specs=[
        pl.BlockSpec((NC, _R, D), lambda i: (0, i, 0)),
        pl.BlockSpec((_R, D), lambda i: (i, 0)),
        pl.BlockSpec((_R, 1), lambda i: (i, 0)),
        pl.BlockSpec((D, D), lambda i: (0, 0)),
        pl.BlockSpec((1, D), lambda i: (0, 0)),
    ],
    out_specs=[
        pl.BlockSpec((_R, D), lambda i: (i, 0)),
        pl.BlockSpec((_R, D), lambda i: (i, 0)),
    ],
    out_shape=[
        jax.ShapeDtypeStruct((N, D), jnp.float32),
        jax.ShapeDtypeStruct((N, D), jnp.float32),
    ],
)


def _head_body(p_ref, v_ref, dinv_ref, wmu_ref, bmu_ref, wlv_ref, blv_ref,
               mu_ref, lv_ref):
    di = dinv_ref[...]
    t = p_ref[0] + p_ref[1]
    a = di * t + (di * di) * v_ref[...]
    mu_ref[...] = jnp.dot(a, wmu_ref[...],
                          preferred_element_type=jnp.float32) + bmu_ref[...]
    lv_ref[...] = jnp.dot(a, wlv_ref[...],
                          preferred_element_type=jnp.float32) + blv_ref[...]


_head_call = pl.pallas_call(
    _head_body,
    grid=(_G,),
    in_specs=[
        pl.BlockSpec((NC, _R, D), lambda i: (0, i, 0)),
        pl.BlockSpec((_R, D), lambda i: (i, 0)),
        pl.BlockSpec((_R, 1), lambda i: (i, 0)),
        pl.BlockSpec((D, DL), lambda i: (0, 0)),
        pl.BlockSpec((1, DL), lambda i: (0, 0)),
        pl.BlockSpec((D, DL), lambda i: (0, 0)),
        pl.BlockSpec((1, DL), lambda i: (0, 0)),
    ],
    out_specs=[
        pl.BlockSpec((_R, DL), lambda i: (i, 0)),
        pl.BlockSpec((_R, DL), lambda i: (i, 0)),
    ],
    out_shape=[
        jax.ShapeDtypeStruct((N, DL), jnp.float32),
        jax.ShapeDtypeStruct((N, DL), jnp.float32),
    ],
)


@jax.jit
def kernel(x, edge_index, W1, b1, W2, b2, Wmu, bmu, Wlv, blv):
    src = edge_index[0]
    dst = edge_index[1]

    pad = E_PAD - E
    pad_ids = lax.iota(jnp.int32, pad)
    src_p = jnp.concatenate([src, pad_ids % N])
    dst_p = jnp.concatenate([dst, N + (pad_ids % 16)])
    src_rows = src_p.reshape(NW, CH, CHUNK)
    dst_rows = dst_p.reshape(NW, CH, CHUNK)

    y1 = _mm1_call(x, W1)
    degp = _deg_kernel(dst_rows)
    dinv, g = _dinv_call(degp, y1)

    p = _agg_kernel(g, src_rows, dst_rows)
    h1, g1 = _layer1_call(p, y1, dinv, b1.reshape(1, D))

    p = _agg_kernel(g1, src_rows, dst_rows)
    h2, g2 = _layer_call(p, h1, dinv, W2, b2.reshape(1, D))

    p = _agg_kernel(g2, src_rows, dst_rows)
    mu, logvar = _head_call(p, h2, dinv, Wmu, bmu.reshape(1, DL),
                            Wlv, blv.reshape(1, DL))
    return (mu, logvar)


# speedup vs baseline: 32.8690x; 1.0000x over previous
import functools

import jax
import jax.numpy as jnp
from jax import lax
from jax.experimental import pallas as pl
from jax.experimental.pallas import tpu as pltpu
from jax.experimental.pallas import tpu_sc as plsc

N = 10000
E = 320000
D = 128
DL = 64

NC = 2
NS = 16
NW = NC * NS

CHUNK = 128
CH = -(-E // (NW * CHUNK))
T = CH * CHUNK
E_PAD = NW * T
ROWS_PER_TILE = 632
N_PAD = NS * ROWS_PER_TILE

_mesh = plsc.VectorSubcoreMesh(core_axis_name="c", subcore_axis_name="s")


N_PAD_DEG = NS * 640


@functools.partial(
    pl.kernel,
    out_type=jax.ShapeDtypeStruct((NC, N_PAD_DEG), jnp.float32),
    mesh=_mesh,
    scratch_types=[
        pltpu.VMEM((CH, CHUNK), jnp.int32),
        pltpu.VMEM((CHUNK,), jnp.float32),
        pltpu.VMEM((128,), jnp.float32),
        pltpu.VMEM_SHARED((N_PAD_DEG,), jnp.float32),
    ],
)
def _deg_kernel(dst_hbm, out_hbm, dst_v, ones_v, zeros_v, acc_s):
    cid = lax.axis_index("c")
    sid = lax.axis_index("s")
    wid = sid * NC + cid
    pltpu.sync_copy(dst_hbm.at[wid], dst_v)

    def fill_body(i, _):
        zeros_v[pl.ds(i * 16, 16)] = jnp.zeros((16,), jnp.float32)
        return 0

    lax.fori_loop(0, 128 // 16, fill_body, 0)

    def fill_ones(i, _):
        ones_v[pl.ds(i * 16, 16)] = jnp.ones((16,), jnp.float32)
        return 0

    lax.fori_loop(0, CHUNK // 16, fill_ones, 0)

    base = sid * 640
    for k in range(5):
        pltpu.sync_copy(zeros_v, acc_s.at[pl.ds(base + k * 128, 128)])
    plsc.subcore_barrier()

    def hist_body(j, _):
        pltpu.sync_copy(ones_v, acc_s.at[dst_v.at[j]], add=True)
        return 0

    lax.fori_loop(0, CH, hist_body, 0)
    plsc.subcore_barrier()

    pltpu.sync_copy(acc_s.at[pl.ds(base, 640)],
                    out_hbm.at[cid].at[pl.ds(base, 640)])


@functools.partial(
    pl.kernel,
    out_type=jax.ShapeDtypeStruct((NC, N_PAD, D), jnp.float32),
    mesh=_mesh,
    scratch_types=[
        pltpu.VMEM((CH, CHUNK), jnp.int32),
        pltpu.VMEM((2, CHUNK), jnp.int32),
        pltpu.VMEM((CHUNK, D), jnp.float32),
        pltpu.VMEM((CHUNK, D), jnp.float32),
        pltpu.VMEM_SHARED((N_PAD, D), jnp.float32),
        pltpu.SemaphoreType.DMA,
        pltpu.SemaphoreType.DMA,
        pltpu.SemaphoreType.DMA,
        pltpu.SemaphoreType.DMA,
    ],
)
def _agg_kernel(g_hbm, src_hbm, dst_hbm, out_hbm,
                src_v, dstb_v, rows0_v, rows1_v, acc_s,
                semg0, semg1, semd0, semd1):
    cid = lax.axis_index("c")
    sid = lax.axis_index("s")
    wid = sid * NC + cid
    pltpu.sync_copy(src_hbm.at[wid], src_v)

    zeros = jnp.zeros((16,), jnp.float32)

    def zero_body(i, _):
        rows0_v[i // (D // 16), pl.ds((i % (D // 16)) * 16, 16)] = zeros
        return 0

    lax.fori_loop(0, CHUNK * (D // 16), zero_body, 0)

    base = sid * ROWS_PER_TILE
    nfull = ROWS_PER_TILE // CHUNK
    rem = ROWS_PER_TILE - nfull * CHUNK
    for k in range(nfull):
        pltpu.async_copy(rows0_v, acc_s.at[pl.ds(base + k * CHUNK, CHUNK)],
                         semg0)
    pltpu.async_copy(rows0_v.at[pl.ds(0, rem)],
                     acc_s.at[pl.ds(base + nfull * CHUNK, rem)], semg1)
    for k in range(nfull):
        pltpu.make_async_copy(
            rows0_v, acc_s.at[pl.ds(base + k * CHUNK, CHUNK)], semg0).wait()
    pltpu.make_async_copy(
        rows0_v.at[pl.ds(0, rem)],
        acc_s.at[pl.ds(base + nfull * CHUNK, rem)], semg1).wait()
    plsc.subcore_barrier()

    bufs = ((rows0_v, semg0, semd0), (rows1_v, semg1, semd1))

    def issue(j, k):
        rows, semg, semd = bufs[k]
        pltpu.async_copy(g_hbm.at[src_v.at[j]], rows, semg)
        pltpu.async_copy(dst_hbm.at[wid].at[j], dstb_v.at[k], semd)

    issue(0, 0)

    def chunk_body(j, _):
        for k in (0, 1):
            @pl.when(j % 2 == k)
            def _():
                rows, semg, semd = bufs[k]

                @pl.when(j + 1 < CH)
                def _():
                    issue(j + 1, 1 - k)

                pltpu.make_async_copy(
                    g_hbm.at[src_v.at[j]], rows, semg).wait()
                pltpu.make_async_copy(
                    dst_hbm.at[wid].at[j], dstb_v.at[k], semd).wait()
                pltpu.sync_copy(rows, acc_s.at[dstb_v.at[k]], add=True)

        return 0

    lax.fori_loop(0, CH, chunk_body, 0)
    plsc.subcore_barrier()

    pltpu.sync_copy(acc_s.at[pl.ds(base, ROWS_PER_TILE)],
                    out_hbm.at[cid].at[pl.ds(base, ROWS_PER_TILE)])


_R = 2048
_G = -(-N // _R)


def _pre_body(degp_ref, x_ref, w_ref, dinv_ref, y_ref, g_ref):
    deg = 1.0 + degp_ref[0, :] + degp_ref[1, :]
    di = lax.rsqrt(deg)[:, None]
    y = jnp.dot(x_ref[...], w_ref[...], preferred_element_type=jnp.float32)
    dinv_ref[...] = di
    y_ref[...] = y
    g_ref[...] = y * di


_pre_call = pl.pallas_call(
    _pre_body,
    grid=(_G,),
    in_specs=[
        pl.BlockSpec((NC, _R), lambda i: (0, i)),
        pl.BlockSpec((_R, D), lambda i: (i, 0)),
        pl.BlockSpec((D, D), lambda i: (0, 0)),
    ],
    out_specs=[
        pl.BlockSpec((_R, 1), lambda i: (i, 0)),
        pl.BlockSpec((_R, D), lambda i: (i, 0)),
        pl.BlockSpec((_R, D), lambda i: (i, 0)),
    ],
    out_shape=[
        jax.ShapeDtypeStruct((N, 1), jnp.float32),
        jax.ShapeDtypeStruct((N, D), jnp.float32),
        jax.ShapeDtypeStruct((N, D), jnp.float32),
    ],
)


def _layer1_body(p_ref, y_ref, dinv_ref, b_ref, h_ref, g_ref):
    di = dinv_ref[...]
    t = p_ref[0] + p_ref[1]
    u = di * t + (di * di) * y_ref[...] + b_ref[...]
    h = jnp.maximum(u, 0.0)
    h_ref[...] = h
    g_ref[...] = h * di


_layer1_call = pl.pallas_call(
    _layer1_body,
    grid=(_G,),
    in_specs=[
        pl.BlockSpec((NC, _R, D), lambda i: (0, i, 0)),
        pl.BlockSpec((_R, D), lambda i: (i, 0)),
        pl.BlockSpec((_R, 1), lambda i: (i, 0)),
        pl.BlockSpec((1, D), lambda i: (0, 0)),
    ],
    out_specs=[
        pl.BlockSpec((_R, D), lambda i: (i, 0)),
        pl.BlockSpec((_R, D), lambda i: (i, 0)),
    ],
    out_shape=[
        jax.ShapeDtypeStruct((N, D), jnp.float32),
        jax.ShapeDtypeStruct((N, D), jnp.float32),
    ],
)


def _layer_body(p_ref, v_ref, dinv_ref, w_ref, b_ref, h_ref, g_ref):
    di = dinv_ref[...]
    t = p_ref[0] + p_ref[1]
    u = di * t + (di * di) * v_ref[...]
    h = jnp.dot(u, w_ref[...], preferred_element_type=jnp.float32)
    h = jnp.maximum(h + b_ref[...], 0.0)
    h_ref[...] = h
    g_ref[...] = h * di


_layer_call = pl.pallas_call(
    _layer_body,
    grid=(_G,),
    in_specs=[
        pl.BlockSpec((NC, _R, D), lambda i: (0, i, 0)),
        pl.BlockSpec((_R, D), lambda i: (i, 0)),
        pl.BlockSpec((_R, 1), lambda i: (i, 0)),
        pl.BlockSpec((D, D), lambda i: (0, 0)),
        pl.BlockSpec((1, D), lambda i: (0, 0)),
    ],
    out_specs=[
        pl.BlockSpec((_R, D), lambda i: (i, 0)),
        pl.BlockSpec((_R, D), lambda i: (i, 0)),
    ],
    out_shape=[
        jax.ShapeDtypeStruct((N, D), jnp.float32),
        jax.ShapeDtypeStruct((N, D), jnp.float32),
    ],
)


def _head_body(p_ref, v_ref, dinv_ref, wmu_ref, bmu_ref, wlv_ref, blv_ref,
               mu_ref, lv_ref):
    di = dinv_ref[...]
    t = p_ref[0] + p_ref[1]
    a = di * t + (di * di) * v_ref[...]
    mu_ref[...] = jnp.dot(a, wmu_ref[...],
                          preferred_element_type=jnp.float32) + bmu_ref[...]
    lv_ref[...] = jnp.dot(a, wlv_ref[...],
                          preferred_element_type=jnp.float32) + blv_ref[...]


_head_call = pl.pallas_call(
    _head_body,
    grid=(_G,),
    in_specs=[
        pl.BlockSpec((NC, _R, D), lambda i: (0, i, 0)),
        pl.BlockSpec((_R, D), lambda i: (i, 0)),
        pl.BlockSpec((_R, 1), lambda i: (i, 0)),
        pl.BlockSpec((D, DL), lambda i: (0, 0)),
        pl.BlockSpec((1, DL), lambda i: (0, 0)),
        pl.BlockSpec((D, DL), lambda i: (0, 0)),
        pl.BlockSpec((1, DL), lambda i: (0, 0)),
    ],
    out_specs=[
        pl.BlockSpec((_R, DL), lambda i: (i, 0)),
        pl.BlockSpec((_R, DL), lambda i: (i, 0)),
    ],
    out_shape=[
        jax.ShapeDtypeStruct((N, DL), jnp.float32),
        jax.ShapeDtypeStruct((N, DL), jnp.float32),
    ],
)


@jax.jit
def kernel(x, edge_index, W1, b1, W2, b2, Wmu, bmu, Wlv, blv):
    src = edge_index[0]
    dst = edge_index[1]

    pad = E_PAD - E
    pad_ids = lax.iota(jnp.int32, pad)
    src_p = jnp.concatenate([src, pad_ids % N])
    dst_p = jnp.concatenate([dst, N + (pad_ids % 16)])
    src_rows = src_p.reshape(NW, CH, CHUNK)
    dst_rows = dst_p.reshape(NW, CH, CHUNK)

    degp = _deg_kernel(dst_rows)
    dinv, y1, g = _pre_call(degp, x, W1)

    p = _agg_kernel(g, src_rows, dst_rows)
    h1, g1 = _layer1_call(p, y1, dinv, b1.reshape(1, D))

    p = _agg_kernel(g1, src_rows, dst_rows)
    h2, g2 = _layer_call(p, h1, dinv, W2, b2.reshape(1, D))

    p = _agg_kernel(g2, src_rows, dst_rows)
    mu, logvar = _head_call(p, h2, dinv, Wmu, bmu.reshape(1, DL),
                            Wlv, blv.reshape(1, DL))
    return (mu, logvar)


# speedup vs baseline: 33.4522x; 1.0177x over previous
import functools

import jax
import jax.numpy as jnp
from jax import lax
from jax.experimental import pallas as pl
from jax.experimental.pallas import tpu as pltpu
from jax.experimental.pallas import tpu_sc as plsc

N = 10000
E = 320000
D = 128
DL = 64

NC = 2
NS = 16
NW = NC * NS

CHUNK = 128
CH = -(-E // (NW * CHUNK))
T = CH * CHUNK
E_PAD = NW * T
ROWS_PER_TILE = 632
N_PAD = NS * ROWS_PER_TILE

_mesh = plsc.VectorSubcoreMesh(core_axis_name="c", subcore_axis_name="s")


N_PAD_DEG = NS * 640


@functools.partial(
    pl.kernel,
    out_type=jax.ShapeDtypeStruct((NC, N_PAD_DEG), jnp.float32),
    mesh=_mesh,
    scratch_types=[
        pltpu.VMEM((CH, CHUNK), jnp.int32),
        pltpu.VMEM((CHUNK,), jnp.float32),
        pltpu.VMEM((128,), jnp.float32),
        pltpu.VMEM_SHARED((N_PAD_DEG,), jnp.float32),
    ],
)
def _deg_kernel(dst_hbm, out_hbm, dst_v, ones_v, zeros_v, acc_s):
    cid = lax.axis_index("c")
    sid = lax.axis_index("s")
    wid = sid * NC + cid
    pltpu.sync_copy(dst_hbm.at[wid], dst_v)

    def fill_body(i, _):
        zeros_v[pl.ds(i * 16, 16)] = jnp.zeros((16,), jnp.float32)
        return 0

    lax.fori_loop(0, 128 // 16, fill_body, 0)

    def fill_ones(i, _):
        ones_v[pl.ds(i * 16, 16)] = jnp.ones((16,), jnp.float32)
        return 0

    lax.fori_loop(0, CHUNK // 16, fill_ones, 0)

    base = sid * 640
    for k in range(5):
        pltpu.sync_copy(zeros_v, acc_s.at[pl.ds(base + k * 128, 128)])
    plsc.subcore_barrier()

    def hist_body(j, _):
        pltpu.sync_copy(ones_v, acc_s.at[dst_v.at[j]], add=True)
        return 0

    lax.fori_loop(0, CH, hist_body, 0)
    plsc.subcore_barrier()

    pltpu.sync_copy(acc_s.at[pl.ds(base, 640)],
                    out_hbm.at[cid].at[pl.ds(base, 640)])


@functools.partial(
    pl.kernel,
    out_type=jax.ShapeDtypeStruct((NC, N_PAD, D), jnp.float32),
    mesh=_mesh,
    scratch_types=[
        pltpu.VMEM((CH, CHUNK), jnp.int32),
        pltpu.VMEM((2, CHUNK), jnp.int32),
        pltpu.VMEM((CHUNK, D), jnp.float32),
        pltpu.VMEM((CHUNK, D), jnp.float32),
        pltpu.VMEM_SHARED((N_PAD, D), jnp.float32),
        pltpu.SemaphoreType.DMA,
        pltpu.SemaphoreType.DMA,
        pltpu.SemaphoreType.DMA,
        pltpu.SemaphoreType.DMA,
        pltpu.SemaphoreType.DMA,
    ],
)
def _agg_kernel(g_hbm, src_hbm, dst_hbm, out_hbm,
                src_v, dstb_v, rows0_v, rows1_v, acc_s,
                semg0, semg1, semd0, semd1, semz):
    cid = lax.axis_index("c")
    sid = lax.axis_index("s")
    wid = sid * NC + cid
    pltpu.sync_copy(src_hbm.at[wid], src_v)

    bufs = ((rows0_v, semg0, semd0), (rows1_v, semg1, semd1))

    def issue(j, k):
        rows, semg, semd = bufs[k]
        pltpu.async_copy(g_hbm.at[src_v.at[j]], rows, semg)
        pltpu.async_copy(dst_hbm.at[wid].at[j], dstb_v.at[k], semd)

    issue(0, 0)

    zeros = jnp.zeros((16,), jnp.float32)

    def zero_body(i, _):
        rows1_v[i // (D // 16), pl.ds((i % (D // 16)) * 16, 16)] = zeros
        return 0

    lax.fori_loop(0, CHUNK * (D // 16), zero_body, 0)

    base = sid * ROWS_PER_TILE
    nfull = ROWS_PER_TILE // CHUNK
    rem = ROWS_PER_TILE - nfull * CHUNK
    for k in range(nfull):
        pltpu.async_copy(rows1_v, acc_s.at[pl.ds(base + k * CHUNK, CHUNK)],
                         semz)
    pltpu.async_copy(rows1_v.at[pl.ds(0, rem)],
                     acc_s.at[pl.ds(base + nfull * CHUNK, rem)], semz)
    for k in range(nfull):
        pltpu.make_async_copy(
            rows1_v, acc_s.at[pl.ds(base + k * CHUNK, CHUNK)], semz).wait()
    pltpu.make_async_copy(
        rows1_v.at[pl.ds(0, rem)],
        acc_s.at[pl.ds(base + nfull * CHUNK, rem)], semz).wait()
    issue(1, 1)
    plsc.subcore_barrier()

    def chunk_body(j, _):
        for k in (0, 1):
            @pl.when(j % 2 == k)
            def _():
                rows, semg, semd = bufs[k]
                pltpu.make_async_copy(
                    g_hbm.at[src_v.at[j]], rows, semg).wait()
                pltpu.make_async_copy(
                    dst_hbm.at[wid].at[j], dstb_v.at[k], semd).wait()
                pltpu.sync_copy(rows, acc_s.at[dstb_v.at[k]], add=True)

                @pl.when(j + 2 < CH)
                def _():
                    issue(j + 2, k)

        return 0

    lax.fori_loop(0, CH, chunk_body, 0)
    plsc.subcore_barrier()

    pltpu.sync_copy(acc_s.at[pl.ds(base, ROWS_PER_TILE)],
                    out_hbm.at[cid].at[pl.ds(base, ROWS_PER_TILE)])


_R = 2048
_G = -(-N // _R)


def _pre_body(degp_ref, x_ref, w_ref, dinv_ref, y_ref, g_ref):
    deg = 1.0 + degp_ref[0, :] + degp_ref[1, :]
    di = lax.rsqrt(deg)[:, None]
    y = jnp.dot(x_ref[...], w_ref[...], preferred_element_type=jnp.float32)
    dinv_ref[...] = di
    y_ref[...] = y
    g_ref[...] = y * di


_pre_call = pl.pallas_call(
    _pre_body,
    grid=(_G,),
    in_specs=[
        pl.BlockSpec((NC, _R), lambda i: (0, i)),
        pl.BlockSpec((_R, D), lambda i: (i, 0)),
        pl.BlockSpec((D, D), lambda i: (0, 0)),
    ],
    out_specs=[
        pl.BlockSpec((_R, 1), lambda i: (i, 0)),
        pl.BlockSpec((_R, D), lambda i: (i, 0)),
        pl.BlockSpec((_R, D), lambda i: (i, 0)),
    ],
    out_shape=[
        jax.ShapeDtypeStruct((N, 1), jnp.float32),
        jax.ShapeDtypeStruct((N, D), jnp.float32),
        jax.ShapeDtypeStruct((N, D), jnp.float32),
    ],
)


def _layer1_body(p_ref, y_ref, dinv_ref, b_ref, h_ref, g_ref):
    di = dinv_ref[...]
    t = p_ref[0] + p_ref[1]
    u = di * t + (di * di) * y_ref[...] + b_ref[...]
    h = jnp.maximum(u, 0.0)
    h_ref[...] = h
    g_ref[...] = h * di


_layer1_call = pl.pallas_call(
    _layer1_body,
    grid=(_G,),
    in_specs=[
        pl.BlockSpec((NC, _R, D), lambda i: (0, i, 0)),
        pl.BlockSpec((_R, D), lambda i: (i, 0)),
        pl.BlockSpec((_R, 1), lambda i: (i, 0)),
        pl.BlockSpec((1, D), lambda i: (0, 0)),
    ],
    out_specs=[
        pl.BlockSpec((_R, D), lambda i: (i, 0)),
        pl.BlockSpec((_R, D), lambda i: (i, 0)),
    ],
    out_shape=[
        jax.ShapeDtypeStruct((N, D), jnp.float32),
        jax.ShapeDtypeStruct((N, D), jnp.float32),
    ],
)


def _layer_body(p_ref, v_ref, dinv_ref, w_ref, b_ref, h_ref, g_ref):
    di = dinv_ref[...]
    t = p_ref[0] + p_ref[1]
    u = di * t + (di * di) * v_ref[...]
    h = jnp.dot(u, w_ref[...], preferred_element_type=jnp.float32)
    h = jnp.maximum(h + b_ref[...], 0.0)
    h_ref[...] = h
    g_ref[...] = h * di


_layer_call = pl.pallas_call(
    _layer_body,
    grid=(_G,),
    in_specs=[
        pl.BlockSpec((NC, _R, D), lambda i: (0, i, 0)),
        pl.BlockSpec((_R, D), lambda i: (i, 0)),
        pl.BlockSpec((_R, 1), lambda i: (i, 0)),
        pl.BlockSpec((D, D), lambda i: (0, 0)),
        pl.BlockSpec((1, D), lambda i: (0, 0)),
    ],
    out_specs=[
        pl.BlockSpec((_R, D), lambda i: (i, 0)),
        pl.BlockSpec((_R, D), lambda i: (i, 0)),
    ],
    out_shape=[
        jax.ShapeDtypeStruct((N, D), jnp.float32),
        jax.ShapeDtypeStruct((N, D), jnp.float32),
    ],
)


def _head_body(p_ref, v_ref, dinv_ref, wmu_ref, bmu_ref, wlv_ref, blv_ref,
               mu_ref, lv_ref):
    di = dinv_ref[...]
    t = p_ref[0] + p_ref[1]
    a = di * t + (di * di) * v_ref[...]
    mu_ref[...] = jnp.dot(a, wmu_ref[...],
                          preferred_element_type=jnp.float32) + bmu_ref[...]
    lv_ref[...] = jnp.dot(a, wlv_ref[...],
                          preferred_element_type=jnp.float32) + blv_ref[...]


_head_call = pl.pallas_call(
    _head_body,
    grid=(_G,),
    in_specs=[
        pl.BlockSpec((NC, _R, D), lambda i: (0, i, 0)),
        pl.BlockSpec((_R, D), lambda i: (i, 0)),
        pl.BlockSpec((_R, 1), lambda i: (i, 0)),
        pl.BlockSpec((D, DL), lambda i: (0, 0)),
        pl.BlockSpec((1, DL), lambda i: (0, 0)),
        pl.BlockSpec((D, DL), lambda i: (0, 0)),
        pl.BlockSpec((1, DL), lambda i: (0, 0)),
    ],
    out_specs=[
        pl.BlockSpec((_R, DL), lambda i: (i, 0)),
        pl.BlockSpec((_R, DL), lambda i: (i, 0)),
    ],
    out_shape=[
        jax.ShapeDtypeStruct((N, DL), jnp.float32),
        jax.ShapeDtypeStruct((N, DL), jnp.float32),
    ],
)


@jax.jit
def kernel(x, edge_index, W1, b1, W2, b2, Wmu, bmu, Wlv, blv):
    src = edge_index[0]
    dst = edge_index[1]

    pad = E_PAD - E
    pad_ids = lax.iota(jnp.int32, pad)
    src_p = jnp.concatenate([src, pad_ids % N])
    dst_p = jnp.concatenate([dst, N + (pad_ids % 16)])
    src_rows = src_p.reshape(NW, CH, CHUNK)
    dst_rows = dst_p.reshape(NW, CH, CHUNK)

    degp = _deg_kernel(dst_rows)
    dinv, y1, g = _pre_call(degp, x, W1)

    p = _agg_kernel(g, src_rows, dst_rows)
    h1, g1 = _layer1_call(p, y1, dinv, b1.reshape(1, D))

    p = _agg_kernel(g1, src_rows, dst_rows)
    h2, g2 = _layer_call(p, h1, dinv, W2, b2.reshape(1, D))

    p = _agg_kernel(g2, src_rows, dst_rows)
    mu, logvar = _head_call(p, h2, dinv, Wmu, bmu.reshape(1, DL),
                            Wlv, blv.reshape(1, DL))
    return (mu, logvar)


# speedup vs baseline: 33.8253x; 1.0112x over previous
import functools

import jax
import jax.numpy as jnp
from jax import lax
from jax.experimental import pallas as pl
from jax.experimental.pallas import tpu as pltpu
from jax.experimental.pallas import tpu_sc as plsc

N = 10000
E = 320000
D = 128
DL = 64

NC = 2
NS = 16
NW = NC * NS

CHUNK = 128
CH = -(-E // (NW * CHUNK))
T = CH * CHUNK
E_PAD = NW * T
ROWS_PER_TILE = 632
N_PAD = NS * ROWS_PER_TILE

_mesh = plsc.VectorSubcoreMesh(core_axis_name="c", subcore_axis_name="s")


N_PAD_DEG = NS * 640


@functools.partial(
    pl.kernel,
    out_type=jax.ShapeDtypeStruct((NC, N_PAD_DEG), jnp.float32),
    mesh=_mesh,
    scratch_types=[
        pltpu.VMEM((CH, CHUNK), jnp.int32),
        pltpu.VMEM((CHUNK,), jnp.float32),
        pltpu.VMEM((128,), jnp.float32),
        pltpu.VMEM_SHARED((N_PAD_DEG,), jnp.float32),
        pltpu.SemaphoreType.DMA,
    ],
)
def _deg_kernel(dst_hbm, out_hbm, dst_v, ones_v, zeros_v, acc_s, semh):
    cid = lax.axis_index("c")
    sid = lax.axis_index("s")
    wid = sid * NC + cid
    pltpu.sync_copy(dst_hbm.at[wid], dst_v)

    def fill_body(i, _):
        zeros_v[pl.ds(i * 16, 16)] = jnp.zeros((16,), jnp.float32)
        return 0

    lax.fori_loop(0, 128 // 16, fill_body, 0)

    def fill_ones(i, _):
        ones_v[pl.ds(i * 16, 16)] = jnp.ones((16,), jnp.float32)
        return 0

    lax.fori_loop(0, CHUNK // 16, fill_ones, 0)

    base = sid * 640
    for k in range(5):
        pltpu.sync_copy(zeros_v, acc_s.at[pl.ds(base + k * 128, 128)])
    plsc.subcore_barrier()

    win = 8

    def fire(j, _):
        pltpu.async_copy(ones_v, acc_s.at[dst_v.at[j]], semh, add=True)
        return 0

    def drain(j, _):
        pltpu.make_async_copy(ones_v, acc_s.at[dst_v.at[j]], semh).wait()
        return 0

    def slide(j, _):
        drain(j - win, 0)
        fire(j, 0)
        return 0

    lax.fori_loop(0, win, fire, 0)
    lax.fori_loop(win, CH, slide, 0)
    lax.fori_loop(CH - win, CH, drain, 0)
    plsc.subcore_barrier()

    pltpu.sync_copy(acc_s.at[pl.ds(base, 640)],
                    out_hbm.at[cid].at[pl.ds(base, 640)])


@functools.partial(
    pl.kernel,
    out_type=jax.ShapeDtypeStruct((NC, N_PAD, D), jnp.float32),
    mesh=_mesh,
    scratch_types=[
        pltpu.VMEM((CH, CHUNK), jnp.int32),
        pltpu.VMEM((2, CHUNK), jnp.int32),
        pltpu.VMEM((CHUNK, D), jnp.float32),
        pltpu.VMEM((CHUNK, D), jnp.float32),
        pltpu.VMEM_SHARED((N_PAD, D), jnp.float32),
        pltpu.SemaphoreType.DMA,
        pltpu.SemaphoreType.DMA,
        pltpu.SemaphoreType.DMA,
        pltpu.SemaphoreType.DMA,
        pltpu.SemaphoreType.DMA,
    ],
)
def _agg_kernel(g_hbm, src_hbm, dst_hbm, out_hbm,
                src_v, dstb_v, rows0_v, rows1_v, acc_s,
                semg0, semg1, semd0, semd1, semz):
    cid = lax.axis_index("c")
    sid = lax.axis_index("s")
    wid = sid * NC + cid
    pltpu.sync_copy(src_hbm.at[wid], src_v)

    bufs = ((rows0_v, semg0, semd0), (rows1_v, semg1, semd1))

    def issue(j, k):
        rows, semg, semd = bufs[k]
        pltpu.async_copy(g_hbm.at[src_v.at[j]], rows, semg)
        pltpu.async_copy(dst_hbm.at[wid].at[j], dstb_v.at[k], semd)

    issue(0, 0)

    zeros = jnp.zeros((16,), jnp.float32)

    def zero_body(i, _):
        rows1_v[i // (D // 16), pl.ds((i % (D // 16)) * 16, 16)] = zeros
        return 0

    lax.fori_loop(0, CHUNK * (D // 16), zero_body, 0)

    base = sid * ROWS_PER_TILE
    nfull = ROWS_PER_TILE // CHUNK
    rem = ROWS_PER_TILE - nfull * CHUNK
    for k in range(nfull):
        pltpu.async_copy(rows1_v, acc_s.at[pl.ds(base + k * CHUNK, CHUNK)],
                         semz)
    pltpu.async_copy(rows1_v.at[pl.ds(0, rem)],
                     acc_s.at[pl.ds(base + nfull * CHUNK, rem)], semz)
    for k in range(nfull):
        pltpu.make_async_copy(
            rows1_v, acc_s.at[pl.ds(base + k * CHUNK, CHUNK)], semz).wait()
    pltpu.make_async_copy(
        rows1_v.at[pl.ds(0, rem)],
        acc_s.at[pl.ds(base + nfull * CHUNK, rem)], semz).wait()
    issue(1, 1)
    plsc.subcore_barrier()

    def chunk_body(j, _):
        for k in (0, 1):
            @pl.when(j % 2 == k)
            def _():
                rows, semg, semd = bufs[k]
                pltpu.make_async_copy(
                    g_hbm.at[src_v.at[j]], rows, semg).wait()
                pltpu.make_async_copy(
                    dst_hbm.at[wid].at[j], dstb_v.at[k], semd).wait()
                pltpu.sync_copy(rows, acc_s.at[dstb_v.at[k]], add=True)

                @pl.when(j + 2 < CH)
                def _():
                    issue(j + 2, k)

        return 0

    lax.fori_loop(0, CH, chunk_body, 0)
    plsc.subcore_barrier()

    pltpu.sync_copy(acc_s.at[pl.ds(base, ROWS_PER_TILE)],
                    out_hbm.at[cid].at[pl.ds(base, ROWS_PER_TILE)])


_R = 2048
_G = -(-N // _R)


def _pre_body(degp_ref, x_ref, w_ref, dinv_ref, y_ref, g_ref):
    deg = 1.0 + degp_ref[0, :] + degp_ref[1, :]
    di = lax.rsqrt(deg)[:, None]
    y = jnp.dot(x_ref[...], w_ref[...], preferred_element_type=jnp.float32)
    dinv_ref[...] = di
    y_ref[...] = y
    g_ref[...] = y * di


_pre_call = pl.pallas_call(
    _pre_body,
    grid=(_G,),
    in_specs=[
        pl.BlockSpec((NC, _R), lambda i: (0, i)),
        pl.BlockSpec((_R, D), lambda i: (i, 0)),
        pl.BlockSpec((D, D), lambda i: (0, 0)),
    ],
    out_specs=[
        pl.BlockSpec((_R, 1), lambda i: (i, 0)),
        pl.BlockSpec((_R, D), lambda i: (i, 0)),
        pl.BlockSpec((_R, D), lambda i: (i, 0)),
    ],
    out_shape=[
        jax.ShapeDtypeStruct((N, 1), jnp.float32),
        jax.ShapeDtypeStruct((N, D), jnp.float32),
        jax.ShapeDtypeStruct((N, D), jnp.float32),
    ],
)


def _layer1_body(p_ref, y_ref, dinv_ref, b_ref, h_ref, g_ref):
    di = dinv_ref[...]
    t = p_ref[0] + p_ref[1]
    u = di * t + (di * di) * y_ref[...] + b_ref[...]
    h = jnp.maximum(u, 0.0)
    h_ref[...] = h
    g_ref[...] = h * di


_layer1_call = pl.pallas_call(
    _layer1_body,
    grid=(_G,),
    in_specs=[
        pl.BlockSpec((NC, _R, D), lambda i: (0, i, 0)),
        pl.BlockSpec((_R, D), lambda i: (i, 0)),
        pl.BlockSpec((_R, 1), lambda i: (i, 0)),
        pl.BlockSpec((1, D), lambda i: (0, 0)),
    ],
    out_specs=[
        pl.BlockSpec((_R, D), lambda i: (i, 0)),
        pl.BlockSpec((_R, D), lambda i: (i, 0)),
    ],
    out_shape=[
        jax.ShapeDtypeStruct((N, D), jnp.float32),
        jax.ShapeDtypeStruct((N, D), jnp.float32),
    ],
)


def _layer_body(p_ref, v_ref, dinv_ref, w_ref, b_ref, h_ref, g_ref):
    di = dinv_ref[...]
    t = p_ref[0] + p_ref[1]
    u = di * t + (di * di) * v_ref[...]
    h = jnp.dot(u, w_ref[...], preferred_element_type=jnp.float32)
    h = jnp.maximum(h + b_ref[...], 0.0)
    h_ref[...] = h
    g_ref[...] = h * di


_layer_call = pl.pallas_call(
    _layer_body,
    grid=(_G,),
    in_specs=[
        pl.BlockSpec((NC, _R, D), lambda i: (0, i, 0)),
        pl.BlockSpec((_R, D), lambda i: (i, 0)),
        pl.BlockSpec((_R, 1), lambda i: (i, 0)),
        pl.BlockSpec((D, D), lambda i: (0, 0)),
        pl.BlockSpec((1, D), lambda i: (0, 0)),
    ],
    out_specs=[
        pl.BlockSpec((_R, D), lambda i: (i, 0)),
        pl.BlockSpec((_R, D), lambda i: (i, 0)),
    ],
    out_shape=[
        jax.ShapeDtypeStruct((N, D), jnp.float32),
        jax.ShapeDtypeStruct((N, D), jnp.float32),
    ],
)


def _head_body(p_ref, v_ref, dinv_ref, wmu_ref, bmu_ref, wlv_ref, blv_ref,
               mu_ref, lv_ref):
    di = dinv_ref[...]
    t = p_ref[0] + p_ref[1]
    a = di * t + (di * di) * v_ref[...]
    mu_ref[...] = jnp.dot(a, wmu_ref[...],
                          preferred_element_type=jnp.float32) + bmu_ref[...]
    lv_ref[...] = jnp.dot(a, wlv_ref[...],
                          preferred_element_type=jnp.float32) + blv_ref[...]


_head_call = pl.pallas_call(
    _head_body,
    grid=(_G,),
    in_specs=[
        pl.BlockSpec((NC, _R, D), lambda i: (0, i, 0)),
        pl.BlockSpec((_R, D), lambda i: (i, 0)),
        pl.BlockSpec((_R, 1), lambda i: (i, 0)),
        pl.BlockSpec((D, DL), lambda i: (0, 0)),
        pl.BlockSpec((1, DL), lambda i: (0, 0)),
        pl.BlockSpec((D, DL), lambda i: (0, 0)),
        pl.BlockSpec((1, DL), lambda i: (0, 0)),
    ],
    out_specs=[
        pl.BlockSpec((_R, DL), lambda i: (i, 0)),
        pl.BlockSpec((_R, DL), lambda i: (i, 0)),
    ],
    out_shape=[
        jax.ShapeDtypeStruct((N, DL), jnp.float32),
        jax.ShapeDtypeStruct((N, DL), jnp.float32),
    ],
)


@jax.jit
def kernel(x, edge_index, W1, b1, W2, b2, Wmu, bmu, Wlv, blv):
    src = edge_index[0]
    dst = edge_index[1]

    pad = E_PAD - E
    pad_ids = lax.iota(jnp.int32, pad)
    src_p = jnp.concatenate([src, pad_ids % N])
    dst_p = jnp.concatenate([dst, N + (pad_ids % 16)])
    src_rows = src_p.reshape(NW, CH, CHUNK)
    dst_rows = dst_p.reshape(NW, CH, CHUNK)

    degp = _deg_kernel(dst_rows)
    dinv, y1, g = _pre_call(degp, x, W1)

    p = _agg_kernel(g, src_rows, dst_rows)
    h1, g1 = _layer1_call(p, y1, dinv, b1.reshape(1, D))

    p = _agg_kernel(g1, src_rows, dst_rows)
    h2, g2 = _layer_call(p, h1, dinv, W2, b2.reshape(1, D))

    p = _agg_kernel(g2, src_rows, dst_rows)
    mu, logvar = _head_call(p, h2, dinv, Wmu, bmu.reshape(1, DL),
                            Wlv, blv.reshape(1, DL))
    return (mu, logvar)
